# Initial kernel scaffold; baseline (speedup 1.0000x reference)
import jax
import jax.numpy as jnp
from jax.experimental import pallas as pl


def kernel(x, idx, ln1_w, ln1_b, ln2_w, ln2_b, W_att, W_conf, W_diff, W_aff, W1, W2):
    raise NotImplementedError("write your pallas kernel here")


# speedup vs baseline: 1.1993x; 1.1993x over previous
import functools

import jax
import jax.numpy as jnp
from jax.experimental import pallas as pl
from jax.experimental.pallas import tpu as pltpu

B, T, C = 1, 2048, 768
E = 8
DFF = 1536


def _ln(x, w, b):
    m = jnp.mean(x, axis=-1, keepdims=True)
    v = jnp.mean(jnp.square(x - m), axis=-1, keepdims=True)
    return (x - m) / jnp.sqrt(v + 1e-5) * w + b


def _route_kernel(x_ref, ln1_w_ref, ln1_b_ref, ln2_w_ref, ln2_b_ref,
                  W_att_ref, W_conf_t_ref, W_diff_ref, W_aff_ref,
                  xa_ref, h_ref, gate_ref):
    x = x_ref[...]
    x_ln = _ln(x, ln1_w_ref[...], ln1_b_ref[...])
    att = jnp.dot(x_ln, W_att_ref[...], preferred_element_type=jnp.float32)
    xa = x + att
    xa_ref[...] = xa
    h = _ln(xa, ln2_w_ref[...], ln2_b_ref[...])
    h_ref[...] = h

    conf = jax.nn.sigmoid(
        jnp.dot(h, W_conf_t_ref[...], preferred_element_type=jnp.float32))
    diff = jax.nn.sigmoid(
        jnp.dot(h, W_diff_ref[...], preferred_element_type=jnp.float32))
    aff = jnp.dot(h, W_aff_ref[...], preferred_element_type=jnp.float32)
    amax = jnp.max(aff, axis=-1, keepdims=True)
    ex = jnp.exp(aff - amax)
    subsidy = ex / jnp.sum(ex, axis=-1, keepdims=True)
    bids = conf * diff + 0.1 * subsidy

    iota = jax.lax.broadcasted_iota(jnp.int32, (T, E), 1)
    m1 = jnp.max(bids, axis=-1, keepdims=True)
    i1 = jnp.min(jnp.where(bids >= m1, iota, E), axis=-1, keepdims=True)
    oh1 = (iota == i1)
    bids2 = jnp.where(oh1, -jnp.inf, bids)
    m2 = jnp.max(bids2, axis=-1, keepdims=True)
    i2 = jnp.min(jnp.where(bids2 >= m2, iota, E), axis=-1, keepdims=True)
    oh2 = (iota == i2)
    u = jnp.exp(m2 - m1)
    w1 = 1.0 / (1.0 + u)
    w2 = u / (1.0 + u)
    gate_ref[...] = jnp.where(oh1, w1, 0.0) + jnp.where(oh2, w2, 0.0)


def _expert_kernel(h_ref, xa_ref, gate_ref, W1_ref, W2_ref, out_ref):
    e = pl.program_id(0)
    h = h_ref[...]
    hid = jnp.dot(h, W1_ref[0], preferred_element_type=jnp.float32)
    hid = jnp.square(jnp.maximum(hid, 0.0))
    eout = jnp.dot(hid, W2_ref[0], preferred_element_type=jnp.float32)
    g = gate_ref[0]
    contrib = eout * g

    @pl.when(e == 0)
    def _():
        out_ref[...] = xa_ref[...] + contrib

    @pl.when(e > 0)
    def _():
        out_ref[...] = out_ref[...] + contrib


@jax.jit
def _run(x, ln1_w, ln1_b, ln2_w, ln2_b, W_att, W_conf, W_diff, W_aff, W1, W2):
    x2 = x.reshape(T, C)
    xa, h, gate = pl.pallas_call(
        _route_kernel,
        out_shape=[
            jax.ShapeDtypeStruct((T, C), jnp.float32),
            jax.ShapeDtypeStruct((T, C), jnp.float32),
            jax.ShapeDtypeStruct((T, E), jnp.float32),
        ],
    )(x2, ln1_w.reshape(1, C), ln1_b.reshape(1, C),
      ln2_w.reshape(1, C), ln2_b.reshape(1, C),
      W_att, W_conf.T, W_diff, W_aff)

    gate_e = gate.T.reshape(E, T, 1)
    out = pl.pallas_call(
        _expert_kernel,
        grid=(E,),
        in_specs=[
            pl.BlockSpec((T, C), lambda e: (0, 0)),
            pl.BlockSpec((T, C), lambda e: (0, 0)),
            pl.BlockSpec((1, T, 1), lambda e: (e, 0, 0)),
            pl.BlockSpec((1, C, DFF), lambda e: (e, 0, 0)),
            pl.BlockSpec((1, DFF, C), lambda e: (e, 0, 0)),
        ],
        out_specs=pl.BlockSpec((T, C), lambda e: (0, 0)),
        out_shape=jax.ShapeDtypeStruct((T, C), jnp.float32),
    )(h, xa, gate_e, W1, W2)
    return out.reshape(B, T, C)


def kernel(x, idx, ln1_w, ln1_b, ln2_w, ln2_b, W_att, W_conf, W_diff, W_aff,
           W1, W2):
    del idx
    return _run(x, ln1_w, ln1_b, ln2_w, ln2_b, W_att, W_conf, W_diff, W_aff,
                W1, W2)
